# baseline (speedup 1.0000x reference)
import functools

import jax
import jax.numpy as jnp
from jax import lax
from jax.experimental import pallas as pl
from jax.experimental.pallas import tpu as pltpu
from jax.experimental.pallas import tpu_sc as plsc

W = 5
D = 64
B = 1024
L = 200
N = B * L


V = 1000000


def _sc_pad(table):
    info = plsc.get_sparse_core_info()
    nw = info.num_cores * info.num_subcores
    chunk = 1600
    n_chunks = V // chunk

    mesh = plsc.VectorSubcoreMesh(core_axis_name="c", subcore_axis_name="s")

    @functools.partial(
        pl.kernel,
        out_type=jax.ShapeDtypeStruct((V, 2 * D), jnp.float32),
        mesh=mesh,
        scratch_types=[pltpu.VMEM((chunk, D), jnp.float32)],
    )
    def pad_kernel(table_hbm, out_hbm, buf_v):
        wid = lax.axis_index("s") * info.num_cores + lax.axis_index("c")

        def body(i, carry):
            c = wid + nw * i

            @pl.when(c < n_chunks)
            def _():
                base = c * chunk
                pltpu.sync_copy(table_hbm.at[pl.ds(base, chunk), :], buf_v)
                pltpu.sync_copy(
                    buf_v, out_hbm.at[pl.ds(base, chunk), pl.ds(0, D)]
                )

            return carry

        lax.fori_loop(0, -(-n_chunks // nw), body, 0)

    return pad_kernel(table)


_CH = 512


def _sc_gather_t(idx_flat, table_pad):
    info = plsc.get_sparse_core_info()
    nw = info.num_cores * info.num_subcores
    n_chunks = 2 * L
    n_iter = -(-n_chunks // nw)

    mesh = plsc.VectorSubcoreMesh(core_axis_name="c", subcore_axis_name="s")

    @functools.partial(
        pl.kernel,
        out_type=jax.ShapeDtypeStruct(((L + W - 1) * D, B), jnp.float32),
        mesh=mesh,
        scratch_types=[
            pltpu.VMEM((_CH,), jnp.int32),
            pltpu.VMEM((_CH, 2 * D), jnp.float32),
            pltpu.VMEM((D, _CH), jnp.float32),
            pltpu.SemaphoreType.DMA,
        ],
        compiler_params=pltpu.CompilerParams(needs_layout_passes=False),
    )
    def gather_kernel(table_hbm, idx_hbm, out_hbm, idx_v, rows_v, slab_v, sem):
        wid = lax.axis_index("s") * info.num_cores + lax.axis_index("c")

        @pl.when(wid < 2 * (W - 1))
        def _():
            def zfill_d(d, carry):
                def zfill_g(g, carry2):
                    slab_v[d, pl.ds(g * 16, 16)] = jnp.zeros((16,), jnp.float32)
                    return carry2

                return lax.fori_loop(0, _CH // 16, zfill_g, carry)

            lax.fori_loop(0, D, zfill_d, 0)
            tz = L + wid // 2
            bz = (wid % 2) * _CH
            pltpu.sync_copy(slab_v, out_hbm.at[pl.ds(tz * D, D), pl.ds(bz, _CH)])

        def body(i, carry):
            c = wid + nw * i

            @pl.when(c < n_chunks)
            def _():
                j = c // 2
                b0 = (c % 2) * _CH
                t = (L - 1) - j
                pltpu.sync_copy(idx_hbm.at[pl.ds(c * _CH, _CH)], idx_v)
                pltpu.async_copy(table_hbm.at[idx_v], rows_v, sem).wait()

                def trans_d(d, carry2):
                    def trans_g(g, carry3):
                        r0 = g * 16
                        rows16 = lax.iota(jnp.int32, 16) + r0
                        vals = plsc.load_gather(
                            rows_v, [rows16, jnp.full((16,), d, jnp.int32)]
                        )
                        slab_v[d, pl.ds(r0, 16)] = vals
                        return carry3

                    return lax.fori_loop(0, _CH // 16, trans_g, carry2)

                lax.fori_loop(0, D, trans_d, 0)
                pltpu.sync_copy(
                    slab_v, out_hbm.at[pl.ds(t * D, D), pl.ds(b0, _CH)]
                )

            return carry

        lax.fori_loop(0, n_iter, body, 0)

    return gather_kernel(table_pad, idx_flat)


def _sc_window(embt2):
    info = plsc.get_sparse_core_info()
    nw = info.num_cores * info.num_subcores
    n_slabs = L + W - 1
    per_w = -(-n_slabs // nw)

    mesh = plsc.VectorSubcoreMesh(core_axis_name="c", subcore_axis_name="s")

    @functools.partial(
        pl.kernel,
        out_type=jax.ShapeDtypeStruct((L, W * D, B), jnp.float32),
        mesh=mesh,
        scratch_types=[pltpu.VMEM((D, B), jnp.float32)],
    )
    def window_kernel(embt_hbm, out_hbm, slab_v):
        wid = lax.axis_index("s") * info.num_cores + lax.axis_index("c")

        def body(i, carry):
            t = wid + nw * i

            @pl.when(t < n_slabs)
            def _():
                pltpu.sync_copy(embt_hbm.at[pl.ds(t * D, D)], slab_v)
                for k in range(W):
                    j = L - 1 - t + k

                    @pl.when((k <= t) & (t - (L - 1) <= k))
                    def _():
                        pltpu.sync_copy(
                            slab_v, out_hbm.at[j, pl.ds(k * D, D)]
                        )

            return carry

        lax.fori_loop(0, per_w, body, 0)

    return window_kernel(embt2)


def kernel(inputs, table):
    table_pad = jnp.pad(table, ((0, 0), (0, 2 * D - D)))
    idxt_flat = inputs.T.reshape(-1).astype(jnp.int32)
    embt2 = _sc_gather_t(idxt_flat, table_pad)
    p = _sc_window(embt2)
    return p.transpose(2, 0, 1)


# speedup vs baseline: 1.0043x; 1.0043x over previous
import functools

import jax
import jax.numpy as jnp
from jax import lax
from jax.experimental import pallas as pl
from jax.experimental.pallas import tpu as pltpu
from jax.experimental.pallas import tpu_sc as plsc

W = 5
D = 64
B = 1024
L = 200
N = B * L


V = 1000000


def _sc_pad(table):
    info = plsc.get_sparse_core_info()
    nw = info.num_cores * info.num_subcores
    chunk = 1600
    n_chunks = V // chunk

    mesh = plsc.VectorSubcoreMesh(core_axis_name="c", subcore_axis_name="s")

    @functools.partial(
        pl.kernel,
        out_type=jax.ShapeDtypeStruct((V, 2 * D), jnp.float32),
        mesh=mesh,
        scratch_types=[pltpu.VMEM((chunk, D), jnp.float32)],
    )
    def pad_kernel(table_hbm, out_hbm, buf_v):
        wid = lax.axis_index("s") * info.num_cores + lax.axis_index("c")

        def body(i, carry):
            c = wid + nw * i

            @pl.when(c < n_chunks)
            def _():
                base = c * chunk
                pltpu.sync_copy(table_hbm.at[pl.ds(base, chunk), :], buf_v)
                pltpu.sync_copy(
                    buf_v, out_hbm.at[pl.ds(base, chunk), pl.ds(0, D)]
                )

            return carry

        lax.fori_loop(0, -(-n_chunks // nw), body, 0)

    return pad_kernel(table)


_CH = 512


def _sc_gather_t(idx_flat, table_pad):
    info = plsc.get_sparse_core_info()
    nw = info.num_cores * info.num_subcores
    n_chunks = 2 * L
    n_iter = -(-n_chunks // nw)

    mesh = plsc.VectorSubcoreMesh(core_axis_name="c", subcore_axis_name="s")

    @functools.partial(
        pl.kernel,
        out_type=jax.ShapeDtypeStruct(((L + W - 1) * D, B), jnp.float32),
        mesh=mesh,
        scratch_types=[
            pltpu.VMEM((_CH,), jnp.int32),
            pltpu.VMEM((_CH, 2 * D), jnp.float32),
            pltpu.VMEM((D, _CH), jnp.float32),
            pltpu.SemaphoreType.DMA,
        ],
        compiler_params=pltpu.CompilerParams(needs_layout_passes=False),
    )
    def gather_kernel(table_hbm, idx_hbm, out_hbm, idx_v, rows_v, slab_v, sem):
        wid = lax.axis_index("s") * info.num_cores + lax.axis_index("c")

        @pl.when(wid < 2 * (W - 1))
        def _():
            def zfill_d(d, carry):
                def zfill_g(g, carry2):
                    slab_v[d, pl.ds(g * 16, 16)] = jnp.zeros((16,), jnp.float32)
                    return carry2

                return lax.fori_loop(0, _CH // 16, zfill_g, carry)

            lax.fori_loop(0, D, zfill_d, 0)
            tz = L + wid // 2
            bz = (wid % 2) * _CH
            pltpu.sync_copy(slab_v, out_hbm.at[pl.ds(tz * D, D), pl.ds(bz, _CH)])

        def body(i, carry):
            c = wid + nw * i

            @pl.when(c < n_chunks)
            def _():
                j = c // 2
                b0 = (c % 2) * _CH
                t = (L - 1) - j
                pltpu.sync_copy(idx_hbm.at[pl.ds(c * _CH, _CH)], idx_v)
                pltpu.async_copy(table_hbm.at[idx_v], rows_v, sem).wait()

                lanes = lax.iota(jnp.int32, 16)

                def trans_d(d, carry2):
                    dcol = jnp.full((16,), d, jnp.int32)
                    for g in range(_CH // 16):
                        vals = plsc.load_gather(rows_v, [lanes + g * 16, dcol])
                        slab_v[d, pl.ds(g * 16, 16)] = vals
                    return carry2

                lax.fori_loop(0, D, trans_d, 0)
                pltpu.sync_copy(
                    slab_v, out_hbm.at[pl.ds(t * D, D), pl.ds(b0, _CH)]
                )

            return carry

        lax.fori_loop(0, n_iter, body, 0)

    return gather_kernel(table_pad, idx_flat)


def _sc_window(embt2):
    info = plsc.get_sparse_core_info()
    nw = info.num_cores * info.num_subcores
    n_slabs = L + W - 1
    per_w = -(-n_slabs // nw)

    mesh = plsc.VectorSubcoreMesh(core_axis_name="c", subcore_axis_name="s")

    @functools.partial(
        pl.kernel,
        out_type=jax.ShapeDtypeStruct((L, W * D, B), jnp.float32),
        mesh=mesh,
        scratch_types=[pltpu.VMEM((D, B), jnp.float32)],
    )
    def window_kernel(embt_hbm, out_hbm, slab_v):
        wid = lax.axis_index("s") * info.num_cores + lax.axis_index("c")

        def body(i, carry):
            t = wid + nw * i

            @pl.when(t < n_slabs)
            def _():
                pltpu.sync_copy(embt_hbm.at[pl.ds(t * D, D)], slab_v)
                for k in range(W):
                    j = L - 1 - t + k

                    @pl.when((k <= t) & (t - (L - 1) <= k))
                    def _():
                        pltpu.sync_copy(
                            slab_v, out_hbm.at[j, pl.ds(k * D, D)]
                        )

            return carry

        lax.fori_loop(0, per_w, body, 0)

    return window_kernel(embt2)


def kernel(inputs, table):
    table_pad = jnp.pad(table, ((0, 0), (0, 2 * D - D)))
    idxt_flat = inputs.T.reshape(-1).astype(jnp.int32)
    embt2 = _sc_gather_t(idxt_flat, table_pad)
    p = _sc_window(embt2)
    return p.transpose(2, 0, 1)
